# baseline (speedup 1.0000x reference)
import functools

import jax
import jax.numpy as jnp
from jax import lax
from jax.experimental import pallas as pl
from jax.experimental.pallas import tpu as pltpu
from jax.experimental.pallas import tpu_sc as plsc

B = 16384
D = 64
NC = 2
NS = 16
NW = NC * NS
BPW = B // NW
GCH = 128
NGCH = BPW // GCH


def _bfly(v):
    lane = lax.iota(jnp.int32, 16)
    for k in (1, 2, 4, 8):
        v = v + v.at[lane ^ k].get(mode="promise_in_bounds")
    return v


def _body(users_hbm, pos_hbm, neg_hbm, utab_hbm, itab_hbm,
          out_p_hbm, out_n_hbm,
          idx_u, idx_p, idx_n, urows, prows, nrows, outp_v, outn_v, sem):
    wid = lax.axis_index("s") * NC + lax.axis_index("c")
    base_tok = wid * BPW

    pltpu.sync_copy(users_hbm.at[pl.ds(base_tok, BPW)], idx_u)
    pltpu.sync_copy(pos_hbm.at[pl.ds(base_tok, BPW)], idx_p)
    pltpu.sync_copy(neg_hbm.at[pl.ds(base_tok, BPW)], idx_n)

    handles = []
    for j in range(NGCH):
        dst = pl.ds(j * GCH, GCH)
        handles.append(pltpu.async_copy(
            utab_hbm.at[idx_u.at[pl.ds(j * GCH, GCH)]], urows.at[dst], sem))
        handles.append(pltpu.async_copy(
            itab_hbm.at[idx_p.at[pl.ds(j * GCH, GCH)]], prows.at[dst], sem))
        handles.append(pltpu.async_copy(
            itab_hbm.at[idx_n.at[pl.ds(j * GCH, GCH)]], nrows.at[dst], sem))
    for h in handles:
        h.wait()

    lane = lax.iota(jnp.int32, 16)

    def group(g, carry):
        base = g * 16
        resp = jnp.zeros((16,), jnp.float32)
        resn = jnp.zeros((16,), jnp.float32)
        for t in range(16):
            i = base + t
            accp = None
            accn = None
            for k in range(D // 16):
                u_k = urows[i, pl.ds(16 * k, 16)]
                p_k = prows[i, pl.ds(16 * k, 16)]
                n_k = nrows[i, pl.ds(16 * k, 16)]
                accp = u_k * p_k if accp is None else accp + u_k * p_k
                accn = u_k * n_k if accn is None else accn + u_k * n_k
            resp = jnp.where(lane == t, _bfly(accp), resp)
            resn = jnp.where(lane == t, _bfly(accn), resn)
        outp_v[pl.ds(base, 16)] = resp
        outn_v[pl.ds(base, 16)] = resn
        return carry

    lax.fori_loop(0, BPW // 16, group, 0)

    pltpu.sync_copy(outp_v, out_p_hbm.at[pl.ds(base_tok, BPW)])
    pltpu.sync_copy(outn_v, out_n_hbm.at[pl.ds(base_tok, BPW)])


@jax.jit
def kernel(users, positives, negatives, user_table, item_table):
    users = users.astype(jnp.int32)
    positives = positives.astype(jnp.int32)
    negatives = negatives.astype(jnp.int32)

    mesh = plsc.VectorSubcoreMesh(core_axis_name="c", subcore_axis_name="s")
    f = pl.kernel(
        _body,
        mesh=mesh,
        compiler_params=pltpu.CompilerParams(
            needs_layout_passes=False, use_tc_tiling_on_sc=False),
        out_type=(
            jax.ShapeDtypeStruct((B,), jnp.float32),
            jax.ShapeDtypeStruct((B,), jnp.float32),
        ),
        scratch_types=[
            pltpu.VMEM((BPW,), jnp.int32),
            pltpu.VMEM((BPW,), jnp.int32),
            pltpu.VMEM((BPW,), jnp.int32),
            pltpu.VMEM((BPW, D), jnp.float32),
            pltpu.VMEM((BPW, D), jnp.float32),
            pltpu.VMEM((BPW, D), jnp.float32),
            pltpu.VMEM((BPW,), jnp.float32),
            pltpu.VMEM((BPW,), jnp.float32),
            pltpu.SemaphoreType.DMA,
        ],
    )
    return f(users, positives, negatives, user_table, item_table)


# speedup vs baseline: 1.9809x; 1.9809x over previous
import functools

import jax
import jax.numpy as jnp
from jax import lax
from jax.experimental import pallas as pl
from jax.experimental.pallas import tpu as pltpu
from jax.experimental.pallas import tpu_sc as plsc

B = 16384
D = 64
NC = 2
NS = 16
NW = NC * NS
BPW = B // NW
CCH = 512
IT_FULL = 1953
IT_TRIPS = 62
UT_FULL = 195
UT_TRIPS = 7
CAP_I = 8192
CAP_U = 4096
MCAP = 2048


def _bfly(v):
    lane = lax.iota(jnp.int32, 16)
    for k in (1, 2, 4, 8):
        v = v + v.at[lane ^ k].get(mode="promise_in_bounds")
    return v


def _body_a(users_hbm, pos_hbm, neg_hbm, utab_hbm, itab_hbm,
            su_hbm, spn_hbm,
            idxblk, wli_idx, wli_tok, wlu_idx, wlu_tok,
            mini_col, mini_tok, ibuf, stage, rowbuf, sem, sem_e):
    wid = lax.axis_index("s") * NC + lax.axis_index("c")
    lane = lax.iota(jnp.int32, 16)

    def scan(src_hbm, tok_offset, wl_i, wl_t, cap, cnt0):
        def blk(bi, cnt):
            pltpu.sync_copy(src_hbm.at[pl.ds(bi * 2048, 2048)], idxblk)

            def q(qi, cnt):
                v = idxblk[pl.ds(qi * 16, 16)]
                m = ((lax.shift_right_logical(v, 9)) & 31) == wid
                tokv = tok_offset + bi * 2048 + qi * 16 + lane
                cc = jnp.minimum(cnt, cap - 16)
                plsc.store_compressed(wl_i.at[pl.ds(cc, 16)], v, mask=m)
                plsc.store_compressed(wl_t.at[pl.ds(cc, 16)], tokv, mask=m)
                return cnt + plsc.all_reduce_population_count(m)[0]

            return lax.fori_loop(0, 128, q, cnt)

        return lax.fori_loop(0, B // 2048, blk, cnt0)

    cnt_i = scan(pos_hbm, 0, wli_idx, wli_tok, CAP_I, jnp.int32(0))
    cnt_i = scan(neg_hbm, B, wli_idx, wli_tok, CAP_I, cnt_i)
    cnt_u = scan(users_hbm, 0, wlu_idx, wlu_tok, CAP_U, jnp.int32(0))

    def process_chunk(c0, width, wl_i, wl_t, wl_cnt, dst_hbm):
        def mt(qi, mc):
            iv = wl_i[pl.ds(qi * 16, 16)]
            tv = wl_t[pl.ds(qi * 16, 16)]
            valid = (qi * 16 + lane) < wl_cnt
            m = valid & (iv >= c0) & (iv < c0 + width)
            mcc = jnp.minimum(mc, MCAP - 16)
            plsc.store_compressed(mini_col.at[pl.ds(mcc, 16)], iv - c0, mask=m)
            plsc.store_compressed(mini_tok.at[pl.ds(mcc, 16)], tv, mask=m)
            return mc + plsc.all_reduce_population_count(m)[0]

        mc = lax.fori_loop(0, (wl_cnt + 15) >> 4, mt, jnp.int32(0))

        def ex(qi, carry):
            d1, d2 = carry
            def dr(_, c):
                pltpu.make_async_copy(
                    dst_hbm.at[pl.ds(0, D)], rowbuf.at[0, 0], sem_e).wait()
                return c

            lax.fori_loop(0, d2, dr, jnp.int32(0))
            par = qi & 1
            colv = mini_col[pl.ds(qi * 16, 16)]
            tokv = mini_tok[pl.ds(qi * 16, 16)]
            valid = (qi * 16 + lane) < mc
            for j in range(D):
                vj = plsc.load_gather(
                    ibuf, [jnp.full((16,), j, jnp.int32), colv], mask=valid)
                plsc.store_scatter(
                    stage, [lane * D + ((j + lane) & (D - 1))], vj, mask=valid)
            rem = mc - qi * 16
            for t in range(16):
                @pl.when(t < rem)
                def _():
                    for k in range(D // 16):
                        uv = plsc.load_gather(
                            stage, [t * D + ((16 * k + lane + t) & (D - 1))])
                        rowbuf[par, t, pl.ds(16 * k, 16)] = uv
                    tok = tokv[t]
                    pltpu.async_copy(
                        rowbuf.at[par, t], dst_hbm.at[pl.ds(tok * D, D)], sem_e)

            return (jnp.minimum(rem, 16), d1)

        d1, d2 = lax.fori_loop(0, (mc + 15) >> 4, ex, (jnp.int32(0), jnp.int32(0)))

        def dr2(_, c):
            pltpu.make_async_copy(
                dst_hbm.at[pl.ds(0, D)], rowbuf.at[0, 0], sem_e).wait()
            return c

        lax.fori_loop(0, d1 + d2, dr2, jnp.int32(0))

    def ichunk(k, carry):
        cid = wid + k * NW

        @pl.when(cid < IT_FULL)
        def _():
            c0 = pl.multiple_of(cid * CCH, CCH)
            pltpu.sync_copy(itab_hbm.at[:, pl.ds(c0, CCH)], ibuf)
            process_chunk(c0, CCH, wli_idx, wli_tok, cnt_i, spn_hbm)

        return carry

    lax.fori_loop(0, IT_TRIPS, ichunk, 0)

    @pl.when(wid == (IT_FULL & 31))
    def _():
        c0t = pl.multiple_of(jnp.int32(IT_FULL) * CCH, 128)
        pltpu.sync_copy(itab_hbm.at[:, pl.ds(c0t, 128)],
                        ibuf.at[:, pl.ds(0, 128)])
        process_chunk(c0t, 128, wli_idx, wli_tok, cnt_i, spn_hbm)

    def uchunk(k, carry):
        cid = wid + k * NW

        @pl.when(cid < UT_FULL)
        def _():
            c0 = pl.multiple_of(cid * CCH, CCH)
            pltpu.sync_copy(utab_hbm.at[:, pl.ds(c0, CCH)], ibuf)
            process_chunk(c0, CCH, wlu_idx, wlu_tok, cnt_u, su_hbm)

        return carry

    lax.fori_loop(0, UT_TRIPS, uchunk, 0)

    @pl.when(wid == (UT_FULL & 31))
    def _():
        c0t = pl.multiple_of(jnp.int32(UT_FULL) * CCH, 128)
        pltpu.sync_copy(utab_hbm.at[:, pl.ds(c0t, 256)],
                        ibuf.at[:, pl.ds(0, 256)])
        process_chunk(c0t, 256, wlu_idx, wlu_tok, cnt_u, su_hbm)


def _body_b(su_hbm, spn_hbm, out_p_hbm, out_n_hbm,
            ub, pb, nb, outp_v, outn_v):
    wid = lax.axis_index("s") * NC + lax.axis_index("c")
    base = wid * BPW
    pltpu.sync_copy(su_hbm.at[pl.ds(base * D, BPW * D)], ub)
    pltpu.sync_copy(spn_hbm.at[pl.ds(base * D, BPW * D)], pb)
    pltpu.sync_copy(spn_hbm.at[pl.ds((B + base) * D, BPW * D)], nb)
    lane = lax.iota(jnp.int32, 16)

    def group(g, carry):
        gb = g * 16
        resp = jnp.zeros((16,), jnp.float32)
        resn = jnp.zeros((16,), jnp.float32)
        for t in range(16):
            o = (gb + t) * D
            accp = None
            accn = None
            for k in range(D // 16):
                u_k = ub[pl.ds(o + 16 * k, 16)]
                p_k = pb[pl.ds(o + 16 * k, 16)]
                n_k = nb[pl.ds(o + 16 * k, 16)]
                accp = u_k * p_k if accp is None else accp + u_k * p_k
                accn = u_k * n_k if accn is None else accn + u_k * n_k
            resp = jnp.where(lane == t, _bfly(accp), resp)
            resn = jnp.where(lane == t, _bfly(accn), resn)
        outp_v[pl.ds(gb, 16)] = resp
        outn_v[pl.ds(gb, 16)] = resn
        return carry

    lax.fori_loop(0, BPW // 16, group, 0)
    pltpu.sync_copy(outp_v, out_p_hbm.at[pl.ds(base, BPW)])
    pltpu.sync_copy(outn_v, out_n_hbm.at[pl.ds(base, BPW)])


@jax.jit
def kernel(users, positives, negatives, user_table, item_table):
    users = users.astype(jnp.int32)
    positives = positives.astype(jnp.int32)
    negatives = negatives.astype(jnp.int32)
    utab = user_table.T
    itab = item_table.T

    mesh = plsc.VectorSubcoreMesh(core_axis_name="c", subcore_axis_name="s")
    cp = pltpu.CompilerParams(needs_layout_passes=False,
                              use_tc_tiling_on_sc=True,
                              disable_bounds_checks=True)
    fa = pl.kernel(
        _body_a,
        mesh=mesh,
        compiler_params=cp,
        out_type=(
            jax.ShapeDtypeStruct((B * D,), jnp.float32),
            jax.ShapeDtypeStruct((2 * B * D,), jnp.float32),
        ),
        scratch_types=[
            pltpu.VMEM((2048,), jnp.int32),
            pltpu.VMEM((CAP_I,), jnp.int32),
            pltpu.VMEM((CAP_I,), jnp.int32),
            pltpu.VMEM((CAP_U,), jnp.int32),
            pltpu.VMEM((CAP_U,), jnp.int32),
            pltpu.VMEM((MCAP,), jnp.int32),
            pltpu.VMEM((MCAP,), jnp.int32),
            pltpu.VMEM((D, CCH), jnp.float32),
            pltpu.VMEM((16 * D,), jnp.float32),
            pltpu.VMEM((2, 16, D), jnp.float32),
            pltpu.SemaphoreType.DMA,
            pltpu.SemaphoreType.DMA,
        ],
    )
    su, spn = fa(users, positives, negatives, utab, itab)

    fb = pl.kernel(
        _body_b,
        mesh=mesh,
        compiler_params=cp,
        out_type=(
            jax.ShapeDtypeStruct((B,), jnp.float32),
            jax.ShapeDtypeStruct((B,), jnp.float32),
        ),
        scratch_types=[
            pltpu.VMEM((BPW * D,), jnp.float32),
            pltpu.VMEM((BPW * D,), jnp.float32),
            pltpu.VMEM((BPW * D,), jnp.float32),
            pltpu.VMEM((BPW,), jnp.float32),
            pltpu.VMEM((BPW,), jnp.float32),
        ],
    )
    return fb(su, spn)


# speedup vs baseline: 2.9751x; 1.5019x over previous
import functools

import jax
import jax.numpy as jnp
from jax import lax
from jax.experimental import pallas as pl
from jax.experimental.pallas import tpu as pltpu
from jax.experimental.pallas import tpu_sc as plsc

B = 16384
D = 64
NC = 2
NS = 16
NW = NC * NS
BPW = B // NW
CCH = 512
IT_FULL = 1953
IT_TRIPS = 62
UT_FULL = 195
UT_TRIPS = 7
CAP_I = 8192
CAP_U = 4096
MCAP = 2048


def _bfly(v):
    lane = lax.iota(jnp.int32, 16)
    for k in (1, 2, 4, 8):
        v = v + v.at[lane ^ k].get(mode="promise_in_bounds")
    return v


def _body_a(users_hbm, pos_hbm, neg_hbm, utab_hbm, itab_hbm,
            su_hbm, spn_hbm,
            idxfull, wli_idx, wli_tok, wlu_idx, wlu_tok,
            mini_col, mini_tok, ibuf0, ibuf1, stage, rowbuf,
            sem0, sem1, sem_e):
    wid = lax.axis_index("s") * NC + lax.axis_index("c")
    lane = lax.iota(jnp.int32, 16)

    def scan(src_hbm, tok_offset, wl_i, wl_t, cap, cnt0):
        pltpu.sync_copy(src_hbm, idxfull)

        def q(qi, cnt):
            v = idxfull[pl.ds(qi * 16, 16)]
            m = ((lax.shift_right_logical(v, 9)) & 31) == wid
            tokv = tok_offset + qi * 16 + lane
            cc = jnp.minimum(cnt, cap - 16)
            plsc.store_compressed(wl_i.at[pl.ds(cc, 16)], v, mask=m)
            plsc.store_compressed(wl_t.at[pl.ds(cc, 16)], tokv, mask=m)
            return cnt + plsc.all_reduce_population_count(m)[0]

        return lax.fori_loop(0, B // 16, q, cnt0)

    cnt_i = scan(pos_hbm, 0, wli_idx, wli_tok, CAP_I, jnp.int32(0))
    cnt_i = scan(neg_hbm, B, wli_idx, wli_tok, CAP_I, cnt_i)
    cnt_u = scan(users_hbm, 0, wlu_idx, wlu_tok, CAP_U, jnp.int32(0))

    def process_chunk(c0, width, wl_i, wl_t, wl_cnt, dst_hbm, ibuf):
        def mt(qi, mc):
            iv = wl_i[pl.ds(qi * 16, 16)]
            tv = wl_t[pl.ds(qi * 16, 16)]
            valid = (qi * 16 + lane) < wl_cnt
            m = valid & (iv >= c0) & (iv < c0 + width)
            mcc = jnp.minimum(mc, MCAP - 16)
            plsc.store_compressed(mini_col.at[pl.ds(mcc, 16)], iv - c0, mask=m)
            plsc.store_compressed(mini_tok.at[pl.ds(mcc, 16)], tv, mask=m)
            return mc + plsc.all_reduce_population_count(m)[0]

        mc = lax.fori_loop(0, (wl_cnt + 15) >> 4, mt, jnp.int32(0))

        def ex(qi, carry):
            d1, d2 = carry
            def dr(_, c):
                pltpu.make_async_copy(
                    dst_hbm.at[pl.ds(0, D)], rowbuf.at[0, 0], sem_e).wait()
                return c

            lax.fori_loop(0, d2, dr, jnp.int32(0))
            par = qi & 1
            colv = mini_col[pl.ds(qi * 16, 16)]
            tokv = mini_tok[pl.ds(qi * 16, 16)]
            valid = (qi * 16 + lane) < mc
            for j in range(D):
                vj = plsc.load_gather(
                    ibuf, [jnp.full((16,), j, jnp.int32), colv], mask=valid)
                plsc.store_scatter(
                    stage, [lane * D + ((j + lane) & (D - 1))], vj, mask=valid)
            rem = mc - qi * 16
            for t in range(16):
                @pl.when(t < rem)
                def _():
                    for k in range(D // 16):
                        uv = plsc.load_gather(
                            stage, [t * D + ((16 * k + lane + t) & (D - 1))])
                        rowbuf[par, t, pl.ds(16 * k, 16)] = uv
                    tok = tokv[t]
                    pltpu.async_copy(
                        rowbuf.at[par, t], dst_hbm.at[pl.ds(tok * D, D)], sem_e)

            return (jnp.minimum(rem, 16), d1)

        d1, d2 = lax.fori_loop(0, (mc + 15) >> 4, ex, (jnp.int32(0), jnp.int32(0)))

        def dr2(_, c):
            pltpu.make_async_copy(
                dst_hbm.at[pl.ds(0, D)], rowbuf.at[0, 0], sem_e).wait()
            return c

        lax.fori_loop(0, d1 + d2, dr2, jnp.int32(0))

    bufs = (ibuf0, ibuf1)
    sems = (sem0, sem1)

    def stream(tab_hbm, nfull, wl_i, wl_t, wl_cnt, dst_hbm, trips):
        def fire(cid, b):
            @pl.when(cid < nfull)
            def _():
                c0 = pl.multiple_of(cid * CCH, CCH)
                pltpu.async_copy(tab_hbm.at[:, pl.ds(c0, CCH)], bufs[b],
                                 sems[b])

        def wait(cid, b):
            @pl.when(cid < nfull)
            def _():
                pltpu.make_async_copy(tab_hbm.at[:, pl.ds(0, CCH)], bufs[b],
                                      sems[b]).wait()

        fire(wid, 0)
        fire(wid + NW, 1)

        def pair(k2, carry):
            for b in range(2):
                k = 2 * k2 + b
                cid = wid + k * NW
                wait(cid, b)

                @pl.when(cid < nfull)
                def _():
                    c0 = pl.multiple_of(cid * CCH, CCH)
                    process_chunk(c0, CCH, wl_i, wl_t, wl_cnt, dst_hbm,
                                  bufs[b])

                fire(wid + (k + 2) * NW, b)
            return carry

        lax.fori_loop(0, (trips + 1) // 2, pair, 0)

    stream(itab_hbm, IT_FULL, wli_idx, wli_tok, cnt_i, spn_hbm, IT_TRIPS)

    @pl.when(wid == (IT_FULL & 31))
    def _():
        c0t = pl.multiple_of(jnp.int32(IT_FULL) * CCH, 128)
        pltpu.sync_copy(itab_hbm.at[:, pl.ds(c0t, 128)],
                        ibuf0.at[:, pl.ds(0, 128)])
        process_chunk(c0t, 128, wli_idx, wli_tok, cnt_i, spn_hbm, ibuf0)

    stream(utab_hbm, UT_FULL, wlu_idx, wlu_tok, cnt_u, su_hbm, UT_TRIPS)

    @pl.when(wid == (UT_FULL & 31))
    def _():
        c0t = pl.multiple_of(jnp.int32(UT_FULL) * CCH, 128)
        pltpu.sync_copy(utab_hbm.at[:, pl.ds(c0t, 256)],
                        ibuf0.at[:, pl.ds(0, 256)])
        process_chunk(c0t, 256, wlu_idx, wlu_tok, cnt_u, su_hbm, ibuf0)


def _body_b(su_hbm, spn_hbm, out_p_hbm, out_n_hbm,
            ub, pb, nb, outp_v, outn_v):
    wid = lax.axis_index("s") * NC + lax.axis_index("c")
    base = wid * BPW
    pltpu.sync_copy(su_hbm.at[pl.ds(base * D, BPW * D)], ub)
    pltpu.sync_copy(spn_hbm.at[pl.ds(base * D, BPW * D)], pb)
    pltpu.sync_copy(spn_hbm.at[pl.ds((B + base) * D, BPW * D)], nb)
    lane = lax.iota(jnp.int32, 16)

    def group(g, carry):
        gb = g * 16
        resp = jnp.zeros((16,), jnp.float32)
        resn = jnp.zeros((16,), jnp.float32)
        for t in range(16):
            o = (gb + t) * D
            accp = None
            accn = None
            for k in range(D // 16):
                u_k = ub[pl.ds(o + 16 * k, 16)]
                p_k = pb[pl.ds(o + 16 * k, 16)]
                n_k = nb[pl.ds(o + 16 * k, 16)]
                accp = u_k * p_k if accp is None else accp + u_k * p_k
                accn = u_k * n_k if accn is None else accn + u_k * n_k
            resp = jnp.where(lane == t, _bfly(accp), resp)
            resn = jnp.where(lane == t, _bfly(accn), resn)
        outp_v[pl.ds(gb, 16)] = resp
        outn_v[pl.ds(gb, 16)] = resn
        return carry

    lax.fori_loop(0, BPW // 16, group, 0)
    pltpu.sync_copy(outp_v, out_p_hbm.at[pl.ds(base, BPW)])
    pltpu.sync_copy(outn_v, out_n_hbm.at[pl.ds(base, BPW)])


@jax.jit
def kernel(users, positives, negatives, user_table, item_table):
    users = users.astype(jnp.int32)
    positives = positives.astype(jnp.int32)
    negatives = negatives.astype(jnp.int32)
    utab = user_table.T
    itab = item_table.T

    mesh = plsc.VectorSubcoreMesh(core_axis_name="c", subcore_axis_name="s")
    cp = pltpu.CompilerParams(needs_layout_passes=False,
                              use_tc_tiling_on_sc=True,
                              disable_bounds_checks=True)
    fa = pl.kernel(
        _body_a,
        mesh=mesh,
        compiler_params=cp,
        out_type=(
            jax.ShapeDtypeStruct((B * D,), jnp.float32),
            jax.ShapeDtypeStruct((2 * B * D,), jnp.float32),
        ),
        scratch_types=[
            pltpu.VMEM((B,), jnp.int32),
            pltpu.VMEM((CAP_I,), jnp.int32),
            pltpu.VMEM((CAP_I,), jnp.int32),
            pltpu.VMEM((CAP_U,), jnp.int32),
            pltpu.VMEM((CAP_U,), jnp.int32),
            pltpu.VMEM((MCAP,), jnp.int32),
            pltpu.VMEM((MCAP,), jnp.int32),
            pltpu.VMEM((D, CCH), jnp.float32),
            pltpu.VMEM((D, CCH), jnp.float32),
            pltpu.VMEM((16 * D,), jnp.float32),
            pltpu.VMEM((2, 16, D), jnp.float32),
            pltpu.SemaphoreType.DMA,
            pltpu.SemaphoreType.DMA,
            pltpu.SemaphoreType.DMA,
        ],
    )
    su, spn = fa(users, positives, negatives, utab, itab)

    fb = pl.kernel(
        _body_b,
        mesh=mesh,
        compiler_params=cp,
        out_type=(
            jax.ShapeDtypeStruct((B,), jnp.float32),
            jax.ShapeDtypeStruct((B,), jnp.float32),
        ),
        scratch_types=[
            pltpu.VMEM((BPW * D,), jnp.float32),
            pltpu.VMEM((BPW * D,), jnp.float32),
            pltpu.VMEM((BPW * D,), jnp.float32),
            pltpu.VMEM((BPW,), jnp.float32),
            pltpu.VMEM((BPW,), jnp.float32),
        ],
    )
    return fb(su, spn)
